# speedup vs baseline: 2.9748x; 2.9748x over previous
import functools

import jax
import jax.numpy as jnp
from jax import lax
from jax.experimental import pallas as pl
from jax.experimental.pallas import tpu as pltpu
from jax.experimental.pallas import tpu_sc as plsc

NW = 32
C = 128


@functools.lru_cache(maxsize=None)
def _build(n, d):
    per_w = n // NW
    nch = per_w // C
    mesh = plsc.VectorSubcoreMesh(core_axis_name="c", subcore_axis_name="s")

    @functools.partial(
        pl.kernel,
        out_type=jax.ShapeDtypeStruct((n, d), jnp.float32),
        mesh=mesh,
        scratch_types=[
            pltpu.VMEM((nch, C), jnp.int32),
            pltpu.VMEM((C, d), jnp.float32),
            pltpu.SemaphoreType.DMA,
        ],
    )
    def gather_kernel(idx_hbm, table_hbm, out_hbm, idx_v, rows_v, gsem):
        wid = lax.axis_index("s") * 2 + lax.axis_index("c")
        base = wid * per_w
        pltpu.sync_copy(idx_hbm.at[wid], idx_v)

        @pl.loop(0, nch)
        def _(g):
            pltpu.async_copy(table_hbm.at[idx_v.at[g]], rows_v, gsem).wait()
            pltpu.sync_copy(rows_v, out_hbm.at[pl.ds(base + g * C, C)])

    return gather_kernel


def kernel(x, table):
    n = x.size
    xf = x.reshape(NW, n // NW // C, C)
    out = _build(n, table.shape[1])(xf, table)
    return out.reshape(x.shape + (table.shape[1],))


# speedup vs baseline: 3.3460x; 1.1248x over previous
import functools

import jax
import jax.numpy as jnp
from jax import lax
from jax.experimental import pallas as pl
from jax.experimental.pallas import tpu as pltpu
from jax.experimental.pallas import tpu_sc as plsc

NW = 32
C = 128


NBUF = 5


@functools.lru_cache(maxsize=None)
def _build(n, d):
    per_w = n // NW
    nch = per_w // C
    assert nch % NBUF == 0
    mesh = plsc.VectorSubcoreMesh(core_axis_name="c", subcore_axis_name="s")

    @functools.partial(
        pl.kernel,
        out_type=jax.ShapeDtypeStruct((n, d), jnp.float32),
        mesh=mesh,
        scratch_types=[
            pltpu.VMEM((nch, C), jnp.int32),
            pltpu.VMEM((NBUF, C, d), jnp.float32),
            pltpu.SemaphoreType.DMA((NBUF,)),
            pltpu.SemaphoreType.DMA((NBUF,)),
        ],
    )
    def gather_kernel(idx_hbm, table_hbm, out_hbm, idx_v, rows_v, gsem, osem):
        wid = lax.axis_index("s") * 2 + lax.axis_index("c")
        base = wid * per_w
        pltpu.sync_copy(idx_hbm.at[wid], idx_v)

        for b in range(NBUF):
            pltpu.async_copy(table_hbm.at[idx_v.at[b]], rows_v.at[b],
                             gsem.at[b])

        @pl.loop(0, nch - NBUF, step=NBUF)
        def _(g):
            for b in range(NBUF):
                cur = g + b
                pltpu.make_async_copy(
                    table_hbm.at[idx_v.at[cur]], rows_v.at[b],
                    gsem.at[b]).wait()
                out_slice = out_hbm.at[pl.ds(base + cur * C, C)]
                pltpu.async_copy(rows_v.at[b], out_slice, osem.at[b])
                pltpu.make_async_copy(rows_v.at[b], out_slice,
                                      osem.at[b]).wait()
                pltpu.async_copy(table_hbm.at[idx_v.at[cur + NBUF]],
                                 rows_v.at[b], gsem.at[b])

        for b in range(NBUF):
            cur = nch - NBUF + b
            pltpu.make_async_copy(
                table_hbm.at[idx_v.at[cur]], rows_v.at[b], gsem.at[b]).wait()
            pltpu.sync_copy(rows_v.at[b],
                            out_hbm.at[pl.ds(base + cur * C, C)])

    return gather_kernel


def kernel(x, table):
    n = x.size
    xf = x.reshape(NW, n // NW // C, C)
    out = _build(n, table.shape[1])(xf, table)
    return out.reshape(x.shape + (table.shape[1],))


# speedup vs baseline: 10.3701x; 3.0992x over previous
import functools

import jax
import jax.numpy as jnp
from jax import lax
from jax.experimental import pallas as pl
from jax.experimental.pallas import tpu as pltpu
from jax.experimental.pallas import tpu_sc as plsc

NW = 32
C = 128


NBUF = 5


@functools.lru_cache(maxsize=None)
def _build(n, d):
    per_w = n // NW
    nch = per_w // C
    assert nch % NBUF == 0
    mesh = plsc.VectorSubcoreMesh(core_axis_name="c", subcore_axis_name="s")

    @functools.partial(
        pl.kernel,
        out_type=jax.ShapeDtypeStruct((n, d), jnp.float32),
        mesh=mesh,
        scratch_types=[
            pltpu.VMEM((nch, C), jnp.int32),
            pltpu.VMEM((NBUF, C, d), jnp.float32),
            pltpu.SemaphoreType.DMA((NBUF,)),
            pltpu.SemaphoreType.DMA((NBUF,)),
        ],
    )
    def gather_kernel(idx_hbm, table_hbm, out_hbm, idx_v, rows_v, gsem, osem):
        wid = lax.axis_index("s") * 2 + lax.axis_index("c")
        base = wid * per_w
        pltpu.sync_copy(idx_hbm.at[wid], idx_v)

        for b in range(NBUF):
            pltpu.async_copy(table_hbm.at[idx_v.at[b]], rows_v.at[b],
                             gsem.at[b])

        @pl.loop(0, nch - NBUF, step=NBUF)
        def _(g):
            for b in range(NBUF):
                cur = g + b
                pltpu.make_async_copy(
                    table_hbm.at[idx_v.at[cur]], rows_v.at[b],
                    gsem.at[b]).wait()
                out_slice = out_hbm.at[pl.ds(base + cur * C, C)]
                pltpu.async_copy(rows_v.at[b], out_slice, osem.at[b])
                pltpu.make_async_copy(rows_v.at[b], out_slice,
                                      osem.at[b]).wait()
                pltpu.async_copy(table_hbm.at[idx_v.at[cur + NBUF]],
                                 rows_v.at[b], gsem.at[b])

        for b in range(NBUF):
            cur = nch - NBUF + b
            pltpu.make_async_copy(
                table_hbm.at[idx_v.at[cur]], rows_v.at[b], gsem.at[b]).wait()
            pltpu.sync_copy(rows_v.at[b],
                            out_hbm.at[pl.ds(base + cur * C, C)])

    return gather_kernel


def kernel(x, table):
    n = x.size
    b, h = x.shape
    d = table.shape[1]
    xf = x.T.reshape(NW, n // NW // C, C)
    out = _build(n, d)(xf, table)
    return out.reshape(h, b, d).transpose(1, 0, 2)


# speedup vs baseline: 10.7313x; 1.0348x over previous
import functools

import jax
import jax.numpy as jnp
from jax import lax
from jax.experimental import pallas as pl
from jax.experimental.pallas import tpu as pltpu
from jax.experimental.pallas import tpu_sc as plsc

NW = 32
C = 128
NBUF = 5
G = 3


@functools.lru_cache(maxsize=None)
def _build(h, b, d):
    nch = h
    mesh = plsc.VectorSubcoreMesh(core_axis_name="c", subcore_axis_name="s")

    @functools.partial(
        pl.kernel,
        out_type=jax.ShapeDtypeStruct((h * b, d), jnp.float32),
        mesh=mesh,
        scratch_types=[
            pltpu.VMEM((nch, C), jnp.int32),
            pltpu.VMEM((NBUF, C, d), jnp.float32),
            pltpu.SemaphoreType.DMA((NBUF,)),
            pltpu.SemaphoreType.DMA((NBUF,)),
        ],
    )
    def gather_kernel(idx_hbm, table_hbm, out_hbm, idx_v, rows_v, gsem, osem):
        wid = lax.axis_index("s") * 2 + lax.axis_index("c")
        col0 = wid * C
        pltpu.sync_copy(idx_hbm.at[:, pl.ds(col0, C)], idx_v)

        def gather_start(k, buf):
            pltpu.async_copy(table_hbm.at[idx_v.at[k]], rows_v.at[buf],
                             gsem.at[buf])

        def gather_wait(k, buf):
            pltpu.make_async_copy(table_hbm.at[idx_v.at[k]], rows_v.at[buf],
                                  gsem.at[buf]).wait()

        def out_ref(k, buf):
            return rows_v.at[buf], out_hbm.at[pl.ds(k * b + col0, C)]

        def store_start(k, buf):
            src, dst = out_ref(k, buf)
            pltpu.async_copy(src, dst, osem.at[buf])

        def store_wait(k, buf):
            src, dst = out_ref(k, buf)
            pltpu.make_async_copy(src, dst, osem.at[buf]).wait()

        for k in range(G):
            gather_start(k, k)

        for k in range(NBUF - G):
            gather_wait(k, k)
            store_start(k, k)
            gather_start(k + G, (k + G) % NBUF)

        @pl.loop(NBUF - G, nch - G, step=NBUF)
        def _(k0):
            for i in range(NBUF):
                buf = (NBUF - G + i) % NBUF
                k = k0 + i
                gather_wait(k, buf)
                store_start(k, buf)
                lbuf = (buf + G) % NBUF
                store_wait(k + G - NBUF, lbuf)
                gather_start(k + G, lbuf)

        for i in range(G):
            k = nch - G + i
            buf = k % NBUF
            gather_wait(k, buf)
            store_start(k, buf)
        for i in range(NBUF):
            k = nch - NBUF + i
            store_wait(k, k % NBUF)

    return gather_kernel


def kernel(x, table):
    b, h = x.shape
    d = table.shape[1]
    assert (h - G) % NBUF == NBUF - G and b == NW * C
    out = _build(h, b, d)(x.T, table)
    return out.reshape(h, b, d).transpose(1, 0, 2)
